# Initial kernel scaffold; baseline (speedup 1.0000x reference)
import jax
import jax.numpy as jnp
from jax.experimental import pallas as pl


def kernel(queries, keys, k):
    raise NotImplementedError("write your pallas kernel here")


# speedup vs baseline: 6.7705x; 6.7705x over previous
import functools

import jax
import jax.numpy as jnp
from jax import lax
from jax.experimental import pallas as pl
from jax.experimental.pallas import tpu as pltpu
from jax.experimental.pallas import tpu_sc as plsc

NEG = -1e30

Q = 4096
D = 128
K = 100000
SEG = 128
KPAD = 102400
S = KPAD // SEG
BQ = 256
BK = 4096
NQ = Q // BQ
NK = KPAD // BK
SB = BK // SEG

NC = 2
NS = 16
NW = NC * NS
RPW = Q // NW
BATCH = 4
NB = RPW // BATCH


def _norm_body(xref, oref):
    x = xref[...]
    xc = x - jnp.mean(x, axis=1, keepdims=True)
    n = jnp.sqrt(jnp.sum(xc * xc, axis=1, keepdims=True))
    oref[...] = xc / (n + 1e-8)


def _normalize(x, br):
    rows = x.shape[0]
    return pl.pallas_call(
        _norm_body,
        grid=(rows // br,),
        in_specs=[pl.BlockSpec((br, D), lambda i: (i, 0))],
        out_specs=pl.BlockSpec((br, D), lambda i: (i, 0)),
        out_shape=jax.ShapeDtypeStruct((rows, D), jnp.float32),
    )(x)


def _mm_body(qref, kref, sref, gref):
    nk = pl.program_id(0)
    sim = lax.dot_general(
        qref[...], kref[...], (((1,), (1,)), ((), ())),
        preferred_element_type=jnp.float32)
    col = lax.broadcasted_iota(jnp.int32, (BQ, BK), 1) + nk * BK
    sim = jnp.where(col < K, sim, NEG)
    sref[...] = sim
    gref[...] = jnp.max(sim.reshape(1, BQ, SB, SEG), axis=3)


def _sims_segmax(qn, kn):
    return pl.pallas_call(
        _mm_body,
        grid=(NK, NQ),
        in_specs=[
            pl.BlockSpec((BQ, D), lambda nk, nq: (nq, 0)),
            pl.BlockSpec((BK, D), lambda nk, nq: (nk, 0)),
        ],
        out_specs=[
            pl.BlockSpec((BQ, BK), lambda nk, nq: (nq, nk)),
            pl.BlockSpec((1, BQ, SB), lambda nk, nq: (nk, nq, 0)),
        ],
        out_shape=[
            jax.ShapeDtypeStruct((Q, KPAD), jnp.float32),
            jax.ShapeDtypeStruct((NK, Q, SB), jnp.float32),
        ],
    )(qn, kn)


def _merge16(tv, ti, cv, ci):
    rv = lax.rev(cv, (0,))
    ri = lax.rev(ci, (0,))
    m = tv >= rv
    hv = jnp.where(m, tv, rv)
    hi = jnp.where(m, ti, ri)
    lv = jnp.where(m, rv, tv)
    li = jnp.where(m, ri, ti)
    hv, hi = plsc.sort_key_val(hv, hi, descending=True)
    lv, li = plsc.sort_key_val(lv, li, descending=True)
    return (hv, hi), (lv, li)


def _insert_chunk(carry, v, vidx):
    t0v, t0i, t1v, t1i, _ = carry
    (t0v, t0i), (lv, li) = _merge16(t0v, t0i, v, vidx)
    (t1v, t1i), _ = _merge16(t1v, t1i, lv, li)
    return t0v, t0i, t1v, t1i, jnp.min(t1v)


def _top32_init():
    return (jnp.full((16,), NEG, jnp.float32), jnp.zeros((16,), jnp.int32),
            jnp.full((16,), NEG, jnp.float32), jnp.zeros((16,), jnp.int32),
            jnp.float32(NEG))


def _sc_body(gmax_hbm, simtab_hbm, vals_hbm, idx_hbm,
             segbuf, gidx, segids, cand, stagev, stagei, sem):
    wid = lax.axis_index("s") * NC + lax.axis_index("c")
    base = wid * RPW

    def batch_body(g, carry_unused):
        q0 = base + g * BATCH
        pltpu.sync_copy(gmax_hbm.at[pl.ds(q0, BATCH)], segbuf)

        for b in range(BATCH):
            qg = q0 + b

            def seg_chunk(i, c):
                v = segbuf[b, pl.ds(i * 16, 16)]
                cm = jnp.max(v)

                def do(c):
                    ci = lax.iota(jnp.int32, 16) + i * 16
                    sv, si = plsc.sort_key_val(v, ci, descending=True)
                    return _insert_chunk(c, sv, si)

                return lax.cond(cm > c[4], do, lambda c: c, c)

            t0v, t0i, t1v, t1i, _ = lax.fori_loop(
                0, S // 16, seg_chunk, _top32_init())
            segids[pl.ds(b * 32, 16)] = t0i
            segids[pl.ds(b * 32 + 16, 16)] = t1i
            gidx[pl.ds(b * 32, 16)] = t0i + qg * S
            gidx[pl.ds(b * 32 + 16, 16)] = t1i + qg * S

        pltpu.async_copy(simtab_hbm.at[gidx], cand, sem).wait()

        for b in range(BATCH):
            def cand_row(j, c):
                r = b * 32 + j
                seg = plsc.load_gather(segids, [jnp.full((16,), r, jnp.int32)])
                for h in range(SEG // 16):
                    v = cand[r, pl.ds(h * 16, 16)]
                    cm = jnp.max(v)
                    kidx = seg * SEG + (lax.iota(jnp.int32, 16) + h * 16)

                    def do(c, v=v, kidx=kidx):
                        sv, si = plsc.sort_key_val(v, kidx, descending=True)
                        return _insert_chunk(c, sv, si)

                    c = lax.cond(cm > c[4], do, lambda c: c, c)
                return c

            f0v, f0i, f1v, f1i, _ = lax.fori_loop(
                0, 32, cand_row, _top32_init())
            stagev[b, pl.ds(0, 16)] = f0v
            stagev[b, pl.ds(16, 16)] = f1v
            stagei[b, pl.ds(0, 16)] = f0i
            stagei[b, pl.ds(16, 16)] = f1i

        pltpu.sync_copy(stagev, vals_hbm.at[pl.ds(q0, BATCH)])
        pltpu.sync_copy(stagei, idx_hbm.at[pl.ds(q0, BATCH)])
        return carry_unused

    lax.fori_loop(0, NB, batch_body, 0)


def _select(segmax, simtab):
    mesh = plsc.VectorSubcoreMesh(
        core_axis_name="c", subcore_axis_name="s",
        num_cores=NC, num_subcores=NS)
    fn = functools.partial(
        pl.kernel, mesh=mesh,
        compiler_params=pltpu.CompilerParams(needs_layout_passes=False),
        out_type=(jax.ShapeDtypeStruct((Q, 32), jnp.float32),
                  jax.ShapeDtypeStruct((Q, 32), jnp.int32)),
        scratch_types=[
            pltpu.VMEM((BATCH, S), jnp.float32),
            pltpu.VMEM((BATCH * 32,), jnp.int32),
            pltpu.VMEM((BATCH * 32,), jnp.int32),
            pltpu.VMEM((BATCH * 32, SEG), jnp.float32),
            pltpu.VMEM((BATCH, 32), jnp.float32),
            pltpu.VMEM((BATCH, 32), jnp.int32),
            pltpu.SemaphoreType.DMA,
        ])(_sc_body)
    return fn(segmax, simtab)


def kernel(queries, keys, k):
    del k
    qn = _normalize(queries, 512)
    kpad = jnp.pad(keys, ((0, KPAD - K), (0, 0)))
    kn = _normalize(kpad, 512)
    sims, segmax3 = _sims_segmax(qn, kn)
    segmax = segmax3.transpose(1, 0, 2).reshape(Q, S)
    simtab = sims.reshape(Q * S, SEG)
    vals, idx = _select(segmax, simtab)
    return vals, idx


# speedup vs baseline: 9.1858x; 1.3567x over previous
import functools

import jax
import jax.numpy as jnp
from jax import lax
from jax.experimental import pallas as pl
from jax.experimental.pallas import tpu as pltpu
from jax.experimental.pallas import tpu_sc as plsc

NEG = -1e30

Q = 4096
D = 128
K = 100000
SEG = 128
KPAD = 102400
S = KPAD // SEG
BQ = 256
BK = 4096
NQ = Q // BQ
NK = KPAD // BK
SB = BK // SEG

NC = 2
NS = 16
NW = NC * NS
RPW = Q // NW
BATCH = 4
NB = RPW // BATCH


def _norm_body(xref, oref):
    x = xref[...]
    xc = x - jnp.mean(x, axis=1, keepdims=True)
    n = jnp.sqrt(jnp.sum(xc * xc, axis=1, keepdims=True))
    oref[...] = xc / (n + 1e-8)


def _normalize(x, br):
    rows = x.shape[0]
    return pl.pallas_call(
        _norm_body,
        grid=(rows // br,),
        in_specs=[pl.BlockSpec((br, D), lambda i: (i, 0))],
        out_specs=pl.BlockSpec((br, D), lambda i: (i, 0)),
        out_shape=jax.ShapeDtypeStruct((rows, D), jnp.float32),
    )(x)


def _mm_body(qref, kref, sref, gref):
    nk = pl.program_id(0)
    sim = lax.dot_general(
        qref[...], kref[...], (((1,), (1,)), ((), ())),
        preferred_element_type=jnp.float32)
    col = lax.broadcasted_iota(jnp.int32, (BQ, BK), 1) + nk * BK
    sim = jnp.where(col < K, sim, NEG)
    sim3 = sim.reshape(BQ, SB, SEG)
    sref[...] = sim3
    gref[...] = jnp.max(sim3, axis=2).reshape(1, BQ, SB)


def _sims_segmax(qn, kn):
    return pl.pallas_call(
        _mm_body,
        grid=(NK, NQ),
        in_specs=[
            pl.BlockSpec((BQ, D), lambda nk, nq: (nq, 0)),
            pl.BlockSpec((BK, D), lambda nk, nq: (nk, 0)),
        ],
        out_specs=[
            pl.BlockSpec((BQ, SB, SEG), lambda nk, nq: (nq, nk, 0)),
            pl.BlockSpec((1, BQ, SB), lambda nk, nq: (nk, nq, 0)),
        ],
        out_shape=[
            jax.ShapeDtypeStruct((Q, S, SEG), jnp.float32),
            jax.ShapeDtypeStruct((NK, Q, SB), jnp.float32),
        ],
    )(qn, kn)


def _merge16(tv, ti, cv, ci):
    rv = lax.rev(cv, (0,))
    ri = lax.rev(ci, (0,))
    m = tv >= rv
    hv = jnp.where(m, tv, rv)
    hi = jnp.where(m, ti, ri)
    lv = jnp.where(m, rv, tv)
    li = jnp.where(m, ri, ti)
    hv, hi = plsc.sort_key_val(hv, hi, descending=True)
    lv, li = plsc.sort_key_val(lv, li, descending=True)
    return (hv, hi), (lv, li)


def _insert_chunk(carry, v, vidx):
    t0v, t0i, t1v, t1i, _ = carry
    (t0v, t0i), (lv, li) = _merge16(t0v, t0i, v, vidx)
    (t1v, t1i), _ = _merge16(t1v, t1i, lv, li)
    return t0v, t0i, t1v, t1i, jnp.min(t1v)


def _top32_init():
    return (jnp.full((16,), NEG, jnp.float32), jnp.zeros((16,), jnp.int32),
            jnp.full((16,), NEG, jnp.float32), jnp.zeros((16,), jnp.int32),
            jnp.float32(NEG))


def _sc_body(gmax_hbm, simtab_hbm, vals_hbm, idx_hbm,
             segbuf, gidx, segids, cand, stagev, stagei, sem):
    wid = lax.axis_index("s") * NC + lax.axis_index("c")
    base = wid * RPW

    def batch_body(g, carry_unused):
        q0 = base + g * BATCH
        pltpu.sync_copy(gmax_hbm.at[pl.ds(q0, BATCH)], segbuf)

        sigmas = []
        for b in range(BATCH):
            qg = q0 + b

            def seg_chunk(i, c):
                v = segbuf[b, pl.ds(i * 16, 16)]
                cm = jnp.max(v)

                def do(c):
                    ci = lax.iota(jnp.int32, 16) + i * 16
                    sv, si = plsc.sort_key_val(v, ci, descending=True)
                    return _insert_chunk(c, sv, si)

                return lax.cond(cm > c[4], do, lambda c: c, c)

            t0v, t0i, t1v, t1i, _ = lax.fori_loop(
                0, S // 16, seg_chunk, _top32_init())
            sigmas.append(jnp.min(t1v))
            segids[pl.ds(b * 32, 16)] = t0i
            segids[pl.ds(b * 32 + 16, 16)] = t1i
            gidx[pl.ds(b * 32, 16)] = t0i + qg * S
            gidx[pl.ds(b * 32 + 16, 16)] = t1i + qg * S

        pltpu.async_copy(simtab_hbm.at[gidx], cand, sem).wait()

        for b in range(BATCH):
            sig = sigmas[b]

            def cand_row(j, c, b=b, sig=sig):
                r = b * 32 + j
                vs = [cand[r, pl.ds(h * 16, 16)] for h in range(SEG // 16)]
                rm = vs[0]
                for h in range(1, SEG // 16):
                    rm = jnp.maximum(rm, vs[h])
                rmax = jnp.max(rm)

                def row_do(c):
                    seg = plsc.load_gather(
                        segids, [jnp.full((16,), r, jnp.int32)])
                    for h in range(SEG // 16):
                        v = vs[h]
                        cm = jnp.max(v)
                        kidx = seg * SEG + (lax.iota(jnp.int32, 16) + h * 16)

                        def do(c, v=v, kidx=kidx):
                            sv, si = plsc.sort_key_val(
                                v, kidx, descending=True)
                            t0v, t0i, t1v, t1i, t = _insert_chunk(c, sv, si)
                            return t0v, t0i, t1v, t1i, jnp.maximum(t, sig)

                        c = lax.cond(cm >= c[4], do, lambda c: c, c)
                    return c

                return lax.cond(rmax >= c[4], row_do, lambda c: c, c)

            t0v_, t0i_, t1v_, t1i_, _ = _top32_init()
            f0v, f0i, f1v, f1i, _ = lax.fori_loop(
                0, 32, cand_row, (t0v_, t0i_, t1v_, t1i_, sig))
            stagev[b, pl.ds(0, 16)] = f0v
            stagev[b, pl.ds(16, 16)] = f1v
            stagei[b, pl.ds(0, 16)] = f0i
            stagei[b, pl.ds(16, 16)] = f1i

        pltpu.sync_copy(stagev, vals_hbm.at[pl.ds(q0, BATCH)])
        pltpu.sync_copy(stagei, idx_hbm.at[pl.ds(q0, BATCH)])
        return carry_unused

    lax.fori_loop(0, NB, batch_body, 0)


def _select(segmax, simtab):
    mesh = plsc.VectorSubcoreMesh(
        core_axis_name="c", subcore_axis_name="s",
        num_cores=NC, num_subcores=NS)
    fn = functools.partial(
        pl.kernel, mesh=mesh,
        compiler_params=pltpu.CompilerParams(needs_layout_passes=False),
        out_type=(jax.ShapeDtypeStruct((Q, 32), jnp.float32),
                  jax.ShapeDtypeStruct((Q, 32), jnp.int32)),
        scratch_types=[
            pltpu.VMEM((BATCH, S), jnp.float32),
            pltpu.VMEM((BATCH * 32,), jnp.int32),
            pltpu.VMEM((BATCH * 32,), jnp.int32),
            pltpu.VMEM((BATCH * 32, SEG), jnp.float32),
            pltpu.VMEM((BATCH, 32), jnp.float32),
            pltpu.VMEM((BATCH, 32), jnp.int32),
            pltpu.SemaphoreType.DMA,
        ])(_sc_body)
    return fn(segmax, simtab)


def kernel(queries, keys, k):
    del k
    qn = _normalize(queries, 512)
    kpad = jnp.pad(keys, ((0, KPAD - K), (0, 0)))
    kn = _normalize(kpad, 512)
    sims3, segmax3 = _sims_segmax(qn, kn)
    segmax = segmax3.transpose(1, 0, 2).reshape(Q, S)
    simtab = sims3.reshape(Q * S, SEG)
    vals, idx = _select(segmax, simtab)
    return vals, idx


# speedup vs baseline: 9.3517x; 1.0181x over previous
import functools

import jax
import jax.numpy as jnp
from jax import lax
from jax.experimental import pallas as pl
from jax.experimental.pallas import tpu as pltpu
from jax.experimental.pallas import tpu_sc as plsc

NEG = -1e30

Q = 4096
D = 128
K = 100000
SEG = 128
KPAD = 102400
S = KPAD // SEG
BQ = 256
BK = 4096
NQ = Q // BQ
NK = KPAD // BK
SB = BK // SEG

NC = 2
NS = 16
NW = NC * NS
RPW = Q // NW
BATCH = 8
NB = RPW // BATCH


def _norm_body(xref, oref):
    x = xref[...]
    xc = x - jnp.mean(x, axis=1, keepdims=True)
    n = jnp.sqrt(jnp.sum(xc * xc, axis=1, keepdims=True))
    oref[...] = xc / (n + 1e-8)


def _normalize(x, br):
    rows = x.shape[0]
    return pl.pallas_call(
        _norm_body,
        grid=(rows // br,),
        in_specs=[pl.BlockSpec((br, D), lambda i: (i, 0))],
        out_specs=pl.BlockSpec((br, D), lambda i: (i, 0)),
        out_shape=jax.ShapeDtypeStruct((rows, D), jnp.float32),
    )(x)


def _mm_body(qref, kref, sref, gref):
    nk = pl.program_id(0)
    sim = lax.dot_general(
        qref[...], kref[...], (((1,), (1,)), ((), ())),
        preferred_element_type=jnp.float32)
    col = lax.broadcasted_iota(jnp.int32, (BQ, BK), 1) + nk * BK
    sim = jnp.where(col < K, sim, NEG)
    sim3 = sim.reshape(BQ, SB, SEG)
    sref[...] = sim3
    gref[...] = jnp.max(sim3, axis=2).reshape(1, BQ, SB)


def _sims_segmax(qn, kn):
    return pl.pallas_call(
        _mm_body,
        grid=(NK, NQ),
        in_specs=[
            pl.BlockSpec((BQ, D), lambda nk, nq: (nq, 0)),
            pl.BlockSpec((BK, D), lambda nk, nq: (nk, 0)),
        ],
        out_specs=[
            pl.BlockSpec((BQ, SB, SEG), lambda nk, nq: (nq, nk, 0)),
            pl.BlockSpec((1, BQ, SB), lambda nk, nq: (nk, nq, 0)),
        ],
        out_shape=[
            jax.ShapeDtypeStruct((Q, S, SEG), jnp.float32),
            jax.ShapeDtypeStruct((NK, Q, SB), jnp.float32),
        ],
    )(qn, kn)


def _merge16(tv, ti, cv, ci):
    rv = lax.rev(cv, (0,))
    ri = lax.rev(ci, (0,))
    m = tv >= rv
    hv = jnp.where(m, tv, rv)
    hi = jnp.where(m, ti, ri)
    lv = jnp.where(m, rv, tv)
    li = jnp.where(m, ri, ti)
    hv, hi = plsc.sort_key_val(hv, hi, descending=True)
    lv, li = plsc.sort_key_val(lv, li, descending=True)
    return (hv, hi), (lv, li)


def _insert_chunk(carry, v, vidx):
    t0v, t0i, t1v, t1i, _ = carry
    (t0v, t0i), (lv, li) = _merge16(t0v, t0i, v, vidx)
    (t1v, t1i), _ = _merge16(t1v, t1i, lv, li)
    return t0v, t0i, t1v, t1i, jnp.min(t1v)


def _top32_init():
    return (jnp.full((16,), NEG, jnp.float32), jnp.zeros((16,), jnp.int32),
            jnp.full((16,), NEG, jnp.float32), jnp.zeros((16,), jnp.int32),
            jnp.float32(NEG))


def _sc_body(gmax_hbm, simtab_hbm, vals_hbm, idx_hbm,
             segbuf_a, segbuf_b, gidx_a, gidx_b, segids_a, segids_b,
             cand_a, cand_b, stagev, stagei, seg_sem, gat_sem):
    wid = lax.axis_index("s") * NC + lax.axis_index("c")
    base = wid * RPW

    def seg_src(g):
        return gmax_hbm.at[pl.ds(base + g * BATCH, BATCH)]

    def fire_seg(g, buf):
        pltpu.async_copy(seg_src(g), buf, seg_sem)

    def wait_seg(g, buf):
        pltpu.make_async_copy(seg_src(g), buf, seg_sem).wait()

    def fire_gat(gidx, cand):
        pltpu.async_copy(simtab_hbm.at[gidx], cand, gat_sem)

    def wait_gat(gidx, cand):
        pltpu.make_async_copy(simtab_hbm.at[gidx], cand, gat_sem).wait()

    def stage1(g, segbuf, gidx, segids):
        sigs = []
        for b in range(BATCH):
            qg = base + g * BATCH + b

            def seg_chunk(i, c):
                v = segbuf[b, pl.ds(i * 16, 16)]
                cm = jnp.max(v)

                def do(c):
                    ci = lax.iota(jnp.int32, 16) + i * 16
                    sv, si = plsc.sort_key_val(v, ci, descending=True)
                    return _insert_chunk(c, sv, si)

                return lax.cond(cm > c[4], do, lambda c: c, c)

            t0v, t0i, t1v, t1i, _ = lax.fori_loop(
                0, S // 16, seg_chunk, _top32_init())
            sigs.append(jnp.min(t1v))
            segids[pl.ds(b * 32, 16)] = t0i
            segids[pl.ds(b * 32 + 16, 16)] = t1i
            gidx[pl.ds(b * 32, 16)] = t0i + qg * S
            gidx[pl.ds(b * 32 + 16, 16)] = t1i + qg * S
        return tuple(sigs)

    def stage2(g, cand, segids, sigs):
        for b in range(BATCH):
            sig = sigs[b]

            def cand_row(j, c, b=b, sig=sig):
                r = b * 32 + j
                vs = [cand[r, pl.ds(h * 16, 16)] for h in range(SEG // 16)]
                rm = vs[0]
                for h in range(1, SEG // 16):
                    rm = jnp.maximum(rm, vs[h])
                rmax = jnp.max(rm)

                def row_do(c):
                    seg = plsc.load_gather(
                        segids, [jnp.full((16,), r, jnp.int32)])
                    for h in range(SEG // 16):
                        v = vs[h]
                        cm = jnp.max(v)
                        kidx = seg * SEG + (lax.iota(jnp.int32, 16) + h * 16)

                        def do(c, v=v, kidx=kidx):
                            sv, si = plsc.sort_key_val(
                                v, kidx, descending=True)
                            t0v, t0i, t1v, t1i, t = _insert_chunk(c, sv, si)
                            return t0v, t0i, t1v, t1i, jnp.maximum(t, sig)

                        c = lax.cond(cm >= c[4], do, lambda c: c, c)
                    return c

                return lax.cond(rmax >= c[4], row_do, lambda c: c, c)

            t0v_, t0i_, t1v_, t1i_, _ = _top32_init()
            f0v, f0i, f1v, f1i, _ = lax.fori_loop(
                0, 32, cand_row, (t0v_, t0i_, t1v_, t1i_, sig))
            rloc = g * BATCH + b
            stagev[rloc, pl.ds(0, 16)] = f0v
            stagev[rloc, pl.ds(16, 16)] = f1v
            stagei[rloc, pl.ds(0, 16)] = f0i
            stagei[rloc, pl.ds(16, 16)] = f1i

    fire_seg(0, segbuf_a)

    def pair_body(i, sig_prev):
        g = i * 2
        wait_seg(g, segbuf_a)
        sigs_a = stage1(g, segbuf_a, gidx_a, segids_a)
        fire_gat(gidx_a, cand_a)
        fire_seg(g + 1, segbuf_b)

        def do_prev(_):
            wait_gat(gidx_b, cand_b)
            stage2(g - 1, cand_b, segids_b, sig_prev)
            return 0

        lax.cond(g > 0, do_prev, lambda _: 0, 0)

        wait_seg(g + 1, segbuf_b)
        sigs_b = stage1(g + 1, segbuf_b, gidx_b, segids_b)
        fire_gat(gidx_b, cand_b)
        gn = jnp.minimum(g + 2, NB - 1)
        fire_seg(gn, segbuf_a)

        wait_gat(gidx_a, cand_a)
        stage2(g, cand_a, segids_a, sigs_a)
        return sigs_b

    sig_last = lax.fori_loop(
        0, NB // 2, pair_body, (jnp.float32(NEG),) * BATCH)

    wait_seg(NB - 1, segbuf_a)
    wait_gat(gidx_b, cand_b)
    stage2(NB - 1, cand_b, segids_b, sig_last)
    pltpu.sync_copy(stagev, vals_hbm.at[pl.ds(base, RPW)])
    pltpu.sync_copy(stagei, idx_hbm.at[pl.ds(base, RPW)])


def _select(segmax, simtab):
    mesh = plsc.VectorSubcoreMesh(
        core_axis_name="c", subcore_axis_name="s",
        num_cores=NC, num_subcores=NS)
    fn = functools.partial(
        pl.kernel, mesh=mesh,
        compiler_params=pltpu.CompilerParams(needs_layout_passes=False),
        out_type=(jax.ShapeDtypeStruct((Q, 32), jnp.float32),
                  jax.ShapeDtypeStruct((Q, 32), jnp.int32)),
        scratch_types=[
            pltpu.VMEM((BATCH, S), jnp.float32),
            pltpu.VMEM((BATCH, S), jnp.float32),
            pltpu.VMEM((BATCH * 32,), jnp.int32),
            pltpu.VMEM((BATCH * 32,), jnp.int32),
            pltpu.VMEM((BATCH * 32,), jnp.int32),
            pltpu.VMEM((BATCH * 32,), jnp.int32),
            pltpu.VMEM((BATCH * 32, SEG), jnp.float32),
            pltpu.VMEM((BATCH * 32, SEG), jnp.float32),
            pltpu.VMEM((RPW, 32), jnp.float32),
            pltpu.VMEM((RPW, 32), jnp.int32),
            pltpu.SemaphoreType.DMA,
            pltpu.SemaphoreType.DMA,
        ])(_sc_body)
    return fn(segmax, simtab)


def kernel(queries, keys, k):
    del k
    qn = _normalize(queries, 512)
    kpad = jnp.pad(keys, ((0, KPAD - K), (0, 0)))
    kn = _normalize(kpad, 512)
    sims3, segmax3 = _sims_segmax(qn, kn)
    segmax = segmax3.transpose(1, 0, 2).reshape(Q, S)
    simtab = sims3.reshape(Q * S, SEG)
    vals, idx = _select(segmax, simtab)
    return vals, idx
